# baseline (speedup 1.0000x reference)
import functools

import jax
import jax.numpy as jnp
from jax import lax
from jax.experimental import pallas as pl
from jax.experimental.pallas import tpu as pltpu
from jax.experimental.pallas import tpu_sc as plsc

_NC = 2
_NS = 16
_NW = _NC * _NS

_CHUNK = 8
_NBUF_T = 2
_NBUF_S = 2
_NSLOT = _NBUF_T + _NBUF_S
_DEPTH = 3


def _sc_body(rows_per_w, emb_hbm, out_hbm, tbuf, sbuf, in_sems, out_sems):
    sid = lax.axis_index("s")
    wid = sid * _NC + lax.axis_index("c")
    base = wid * rows_per_w
    nchunks = rows_per_w // _CHUNK

    def buf_at(slot):
        if slot < _NBUF_T:
            return tbuf.at[slot]
        return sbuf.at[sid, slot - _NBUF_T]

    def in_copy(c, slot):
        return pltpu.make_async_copy(
            emb_hbm.at[pl.ds(base + c * _CHUNK, _CHUNK)], buf_at(slot),
            in_sems.at[slot])

    def out_copy(c, slot):
        return pltpu.make_async_copy(
            buf_at(slot), out_hbm.at[pl.ds(base + c * _CHUNK, _CHUNK)],
            out_sems.at[slot])

    for c in range(min(_DEPTH, nchunks)):
        in_copy(c, c % _NSLOT).start()
    for c in range(nchunks):
        slot = c % _NSLOT
        p = c + _DEPTH
        if p < nchunks:
            sp = p % _NSLOT
            if p - _NSLOT >= 0:
                out_copy(p - _NSLOT, sp).wait()
            in_copy(p, sp).start()
        in_copy(c, slot).wait()
        out_copy(c, slot).start()
    for c in range(max(0, nchunks - _NSLOT), nchunks):
        out_copy(c, c % _NSLOT).wait()


def kernel(x, embeddings):
    seq_len = x.shape[1]
    hidden = embeddings.shape[1]
    rows_per_w = seq_len // _NW
    mesh = plsc.VectorSubcoreMesh(
        core_axis_name="c", subcore_axis_name="s",
        num_cores=_NC, num_subcores=_NS)
    sc_copy = functools.partial(
        pl.kernel,
        mesh=mesh,
        out_type=jax.ShapeDtypeStruct((seq_len, hidden), embeddings.dtype),
        scratch_types=[
            pltpu.VMEM((_NBUF_T, _CHUNK, hidden), embeddings.dtype),
            pltpu.VMEM_SHARED((_NS, _NBUF_S, _CHUNK, hidden), embeddings.dtype),
            pltpu.SemaphoreType.DMA((_NSLOT,)),
            pltpu.SemaphoreType.DMA((_NSLOT,)),
        ],
    )(functools.partial(_sc_body, rows_per_w))
    return sc_copy(embeddings[:seq_len])


# speedup vs baseline: 1.0294x; 1.0294x over previous
import functools

import jax
import jax.numpy as jnp
from jax import lax
from jax.experimental import pallas as pl
from jax.experimental.pallas import tpu as pltpu
from jax.experimental.pallas import tpu_sc as plsc

_NC = 2
_NS = 16
_WPS = 8
_NW = _NC * _WPS

_CHUNK = 16
_NBUF = 3
_DEPTH = 2


def _sc_body(rows_per_w, emb_hbm, out_hbm, sbuf, in_sems, out_sems):
    sid = lax.axis_index("s")
    wid = sid * _NC + lax.axis_index("c")
    base = wid * rows_per_w
    nchunks = rows_per_w // _CHUNK

    def in_copy(c, b):
        return pltpu.make_async_copy(
            emb_hbm.at[pl.ds(base + c * _CHUNK, _CHUNK)], sbuf.at[sid, b],
            in_sems.at[b])

    def out_copy(c, b):
        return pltpu.make_async_copy(
            sbuf.at[sid, b], out_hbm.at[pl.ds(base + c * _CHUNK, _CHUNK)],
            out_sems.at[b])

    @pl.when(sid < _WPS)
    def _():
        for c in range(min(_DEPTH, nchunks)):
            in_copy(c, c % _NBUF).start()
        for c in range(nchunks):
            b = c % _NBUF
            p = c + _DEPTH
            if p < nchunks:
                bp = p % _NBUF
                if p - _NBUF >= 0:
                    out_copy(p - _NBUF, bp).wait()
                in_copy(p, bp).start()
            in_copy(c, b).wait()
            out_copy(c, b).start()
        for c in range(max(0, nchunks - _NBUF), nchunks):
            out_copy(c, c % _NBUF).wait()


def kernel(x, embeddings):
    seq_len = x.shape[1]
    hidden = embeddings.shape[1]
    rows_per_w = seq_len // _NW
    mesh = plsc.VectorSubcoreMesh(
        core_axis_name="c", subcore_axis_name="s",
        num_cores=_NC, num_subcores=_NS)
    sc_copy = functools.partial(
        pl.kernel,
        mesh=mesh,
        out_type=jax.ShapeDtypeStruct((seq_len, hidden), embeddings.dtype),
        scratch_types=[
            pltpu.VMEM_SHARED((_WPS, _NBUF, _CHUNK, hidden), embeddings.dtype),
            pltpu.SemaphoreType.DMA((_NBUF,)),
            pltpu.SemaphoreType.DMA((_NBUF,)),
        ],
    )(functools.partial(_sc_body, rows_per_w))
    return sc_copy(embeddings[:seq_len])


# speedup vs baseline: 1.0594x; 1.0291x over previous
import functools

import jax
import jax.numpy as jnp
from jax import lax
from jax.experimental import pallas as pl
from jax.experimental.pallas import tpu as pltpu
from jax.experimental.pallas import tpu_sc as plsc

_NC = 2

_CHUNK = 64
_NBUF = 6
_DEPTH = 3


def _sc_body(rows_per_w, emb_hbm, out_hbm, sbuf, in_sems, out_sems):
    base = lax.axis_index("c") * rows_per_w
    nchunks = rows_per_w // _CHUNK

    def in_copy(c, b):
        return pltpu.make_async_copy(
            emb_hbm.at[pl.ds(base + c * _CHUNK, _CHUNK)], sbuf.at[b],
            in_sems.at[b])

    def out_copy(c, b):
        return pltpu.make_async_copy(
            sbuf.at[b], out_hbm.at[pl.ds(base + c * _CHUNK, _CHUNK)],
            out_sems.at[b])

    for c in range(min(_DEPTH, nchunks)):
        in_copy(c, c % _NBUF).start()
    for c in range(nchunks):
        b = c % _NBUF
        p = c + _DEPTH
        if p < nchunks:
            bp = p % _NBUF
            if p - _NBUF >= 0:
                out_copy(p - _NBUF, bp).wait()
            in_copy(p, bp).start()
        in_copy(c, b).wait()
        out_copy(c, b).start()
    for c in range(max(0, nchunks - _NBUF), nchunks):
        out_copy(c, c % _NBUF).wait()


def kernel(x, embeddings):
    seq_len = x.shape[1]
    hidden = embeddings.shape[1]
    rows_per_w = seq_len // _NC
    mesh = plsc.ScalarSubcoreMesh(axis_name="c", num_cores=_NC)
    sc_copy = functools.partial(
        pl.kernel,
        mesh=mesh,
        out_type=jax.ShapeDtypeStruct((seq_len, hidden), embeddings.dtype),
        scratch_types=[
            pltpu.VMEM_SHARED((_NBUF, _CHUNK, hidden), embeddings.dtype),
            pltpu.SemaphoreType.DMA((_NBUF,)),
            pltpu.SemaphoreType.DMA((_NBUF,)),
        ],
    )(functools.partial(_sc_body, rows_per_w))
    return sc_copy(embeddings[:seq_len])
